# Initial kernel scaffold; baseline (speedup 1.0000x reference)
import jax
import jax.numpy as jnp
from jax.experimental import pallas as pl


def kernel(text_ids, text_emb):
    raise NotImplementedError("write your pallas kernel here")


# speedup vs baseline: 1.5698x; 1.5698x over previous
import functools

import jax
import jax.numpy as jnp
from jax import lax
from jax.experimental import pallas as pl
from jax.experimental.pallas import tpu as pltpu
from jax.experimental.pallas import tpu_sc as plsc

D_MODEL = 1024
N_TOKENS = 4 * 4096

_info = plsc.get_sparse_core_info()
NUM_CORES = _info.num_cores
NUM_SUBCORES = _info.num_subcores
NW = NUM_CORES * NUM_SUBCORES
ROWS_PER_W = N_TOKENS // NW
CHUNK = 32
NCHUNKS = ROWS_PER_W // CHUNK
NBUF = 2


def _make_gather():
    mesh = plsc.VectorSubcoreMesh(core_axis_name="c", subcore_axis_name="s")

    @functools.partial(
        pl.kernel,
        mesh=mesh,
        out_type=jax.ShapeDtypeStruct((N_TOKENS, D_MODEL), jnp.float32),
        scratch_types=[
            pltpu.VMEM((ROWS_PER_W,), jnp.int32),
            pltpu.VMEM((CHUNK, D_MODEL), jnp.float32),
            pltpu.VMEM((CHUNK, D_MODEL), jnp.float32),
            pltpu.SemaphoreType.DMA,
            pltpu.SemaphoreType.DMA,
            pltpu.SemaphoreType.DMA,
        ],
    )
    def gather_kernel(ids_hbm, table_hbm, out_hbm, idx_v, buf0, buf1,
                      gsem, osem0, osem1):
        wid = lax.axis_index("s") * NUM_CORES + lax.axis_index("c")
        base = wid * ROWS_PER_W
        pltpu.sync_copy(ids_hbm.at[pl.ds(base, ROWS_PER_W)], idx_v)

        bufs = (buf0, buf1)
        osems = (osem0, osem1)
        out_descs = [None] * NBUF
        for i in range(NCHUNKS):
            j = i % NBUF
            if out_descs[j] is not None:
                out_descs[j].wait()
            pltpu.async_copy(
                table_hbm.at[idx_v.at[pl.ds(i * CHUNK, CHUNK)]],
                bufs[j], gsem).wait()
            out_descs[j] = pltpu.async_copy(
                bufs[j], out_hbm.at[pl.ds(base + i * CHUNK, CHUNK)],
                osems[j])
        for j in range(NBUF):
            if out_descs[j] is not None:
                out_descs[j].wait()

    return gather_kernel


_gather = _make_gather()


def kernel(text_ids, text_emb):
    ids = text_ids.reshape(-1).astype(jnp.int32)
    out = _gather(ids, text_emb)
    return out.reshape(text_ids.shape[0], text_ids.shape[1], D_MODEL)


# speedup vs baseline: 1.6593x; 1.0570x over previous
import functools

import jax
import jax.numpy as jnp
from jax import lax
from jax.experimental import pallas as pl
from jax.experimental.pallas import tpu as pltpu
from jax.experimental.pallas import tpu_sc as plsc

D_MODEL = 1024
N_TOKENS = 4 * 4096

_info = plsc.get_sparse_core_info()
NUM_CORES = _info.num_cores
NUM_SUBCORES = _info.num_subcores
NW = NUM_CORES * NUM_SUBCORES
ROWS_PER_W = N_TOKENS // NW
CHUNK = 32
NCHUNKS = ROWS_PER_W // CHUNK
NBUF = 3


def _make_gather():
    mesh = plsc.VectorSubcoreMesh(core_axis_name="c", subcore_axis_name="s")

    scratch = [pltpu.VMEM((ROWS_PER_W,), jnp.int32)]
    scratch += [pltpu.VMEM((CHUNK, D_MODEL), jnp.float32)] * NBUF
    scratch += [pltpu.SemaphoreType.DMA] * (2 * NBUF)

    @functools.partial(
        pl.kernel,
        mesh=mesh,
        out_type=jax.ShapeDtypeStruct((N_TOKENS, D_MODEL), jnp.float32),
        scratch_types=scratch,
    )
    def gather_kernel(ids_hbm, table_hbm, out_hbm, idx_v, *rest):
        bufs = rest[:NBUF]
        gsems = rest[NBUF:2 * NBUF]
        osems = rest[2 * NBUF:]
        wid = lax.axis_index("s") * NUM_CORES + lax.axis_index("c")
        base = wid * ROWS_PER_W
        pltpu.sync_copy(ids_hbm.at[pl.ds(base, ROWS_PER_W)], idx_v)

        g_descs = [None] * NBUF
        o_descs = [None] * NBUF

        def start_gather(i):
            j = i % NBUF
            if o_descs[j] is not None:
                o_descs[j].wait()
            g_descs[j] = pltpu.async_copy(
                table_hbm.at[idx_v.at[pl.ds(i * CHUNK, CHUNK)]],
                bufs[j], gsems[j])

        def start_scatter(i):
            j = i % NBUF
            g_descs[j].wait()
            o_descs[j] = pltpu.async_copy(
                bufs[j], out_hbm.at[pl.ds(base + i * CHUNK, CHUNK)],
                osems[j])

        for i in range(NCHUNKS):
            start_gather(i)
            if i >= NBUF - 1:
                start_scatter(i - (NBUF - 1))
        for i in range(NCHUNKS - (NBUF - 1), NCHUNKS):
            start_scatter(i)
        for j in range(NBUF):
            o_descs[j].wait()

    return gather_kernel


_gather = _make_gather()


def kernel(text_ids, text_emb):
    ids = text_ids.reshape(-1).astype(jnp.int32)
    out = _gather(ids, text_emb)
    return out.reshape(text_ids.shape[0], text_ids.shape[1], D_MODEL)


# speedup vs baseline: 1.6668x; 1.0045x over previous
import functools

import jax
import jax.numpy as jnp
from jax import lax
from jax.experimental import pallas as pl
from jax.experimental.pallas import tpu as pltpu
from jax.experimental.pallas import tpu_sc as plsc

D_MODEL = 1024
N_TOKENS = 4 * 4096

_info = plsc.get_sparse_core_info()
NUM_CORES = _info.num_cores
NUM_SUBCORES = _info.num_subcores
NW = NUM_CORES * NUM_SUBCORES
ROWS_PER_W = N_TOKENS // NW
CHUNK = 16
NCHUNKS = ROWS_PER_W // CHUNK
NBUF = 6


def _make_gather():
    mesh = plsc.VectorSubcoreMesh(core_axis_name="c", subcore_axis_name="s")

    scratch = [pltpu.VMEM((ROWS_PER_W,), jnp.int32)]
    scratch += [pltpu.VMEM((CHUNK, D_MODEL), jnp.float32)] * NBUF
    scratch += [pltpu.SemaphoreType.DMA] * (2 * NBUF)

    @functools.partial(
        pl.kernel,
        mesh=mesh,
        out_type=jax.ShapeDtypeStruct((N_TOKENS, D_MODEL), jnp.float32),
        scratch_types=scratch,
    )
    def gather_kernel(ids_hbm, table_hbm, out_hbm, idx_v, *rest):
        bufs = rest[:NBUF]
        gsems = rest[NBUF:2 * NBUF]
        osems = rest[2 * NBUF:]
        wid = lax.axis_index("s") * NUM_CORES + lax.axis_index("c")
        base = wid * ROWS_PER_W
        pltpu.sync_copy(ids_hbm.at[pl.ds(base, ROWS_PER_W)], idx_v)

        g_descs = [None] * NBUF
        o_descs = [None] * NBUF

        def start_gather(i):
            j = i % NBUF
            if o_descs[j] is not None:
                o_descs[j].wait()
            g_descs[j] = pltpu.async_copy(
                table_hbm.at[idx_v.at[pl.ds(i * CHUNK, CHUNK)]],
                bufs[j], gsems[j])

        def start_scatter(i):
            j = i % NBUF
            g_descs[j].wait()
            o_descs[j] = pltpu.async_copy(
                bufs[j], out_hbm.at[pl.ds(base + i * CHUNK, CHUNK)],
                osems[j])

        for i in range(NCHUNKS):
            start_gather(i)
            if i >= NBUF - 1:
                start_scatter(i - (NBUF - 1))
        for i in range(NCHUNKS - (NBUF - 1), NCHUNKS):
            start_scatter(i)
        for j in range(NBUF):
            o_descs[j].wait()

    return gather_kernel


_gather = _make_gather()


def kernel(text_ids, text_emb):
    ids = text_ids.reshape(-1).astype(jnp.int32)
    out = _gather(ids, text_emb)
    return out.reshape(text_ids.shape[0], text_ids.shape[1], D_MODEL)
